# baseline (speedup 1.0000x reference)
import jax
import jax.numpy as jnp
from jax import lax
from jax.experimental import pallas as pl
from jax.experimental.pallas import tpu as pltpu
from jax.experimental.pallas import tpu_sc as plsc

N = 10000
D = 128
E = 320000
EPS = 1e-5

NC = 2
NS = 16
NB = NC * NS
BR = 320
NP2 = BR * NB
SCHUNK = 8000
NSC = E // SCHUNK
FBUF = 8448
CAP = E + 4096
LCH = 128
SBE = 2048
PADBASE = (BR - 1) * NB

_MESH = plsc.VectorSubcoreMesh(core_axis_name="c", subcore_axis_name="s",
                               num_cores=NC, num_subcores=NS)


def _sc_bucket_body(packed, lists, counts, stage_v, fbuf, cnt_v):
    c = lax.axis_index("c")
    s = lax.axis_index("s")
    b = c * NS + s
    lbase = pl.multiple_of(b * CAP, 8)

    def outer(sc, carry):
        off, goff = carry
        pltpu.sync_copy(packed.at[pl.ds(sc * SCHUNK, SCHUNK)], stage_v)

        def scan(i, o):
            p = stage_v[pl.ds(i * 16, 16)]
            msk = (p & 31) == b
            plsc.store_compressed(fbuf.at[pl.ds(o, 16)], p, mask=msk)
            return o + plsc.all_reduce_population_count(msk)[0]

        off = lax.fori_loop(0, SCHUNK // 16, scan, off)
        nf = off // 128
        r0 = nf * 128

        def fl(k, carry2):
            dst0 = pl.multiple_of(lbase + goff + k * 128, 8)
            pltpu.sync_copy(fbuf.at[pl.ds(k * 128, 128)],
                            lists.at[pl.ds(dst0, 128)])
            return carry2

        lax.fori_loop(0, nf, fl, 0)

        def mv(j, carry2):
            v = fbuf[pl.ds(r0 + j * 16, 16)]
            fbuf[pl.ds(j * 16, 16)] = v
            return carry2

        lax.fori_loop(0, 9, mv, 0)
        return (off - r0, goff + r0)

    off, goff = lax.fori_loop(0, NSC, outer, (0, 0))

    padv = jnp.full((16,), PADBASE, jnp.int32) + b
    fullm = jnp.ones((16,), jnp.bool_)

    def padstep(k, o):
        plsc.store_compressed(fbuf.at[pl.ds(o, 16)], padv, mask=fullm)
        return o + 16

    lax.fori_loop(0, 16, padstep, off)
    total = goff // 128 + (off + 127) // 128
    ncnk = total + (total & 1)
    tail_rows = ncnk - goff // 128

    def fflush(k, carry):
        dst0 = pl.multiple_of(lbase + goff + k * 128, 8)
        pltpu.sync_copy(fbuf.at[pl.ds(k * 128, 128)],
                        lists.at[pl.ds(dst0, 128)])
        return carry

    lax.fori_loop(0, tail_rows, fflush, 0)
    cnt_v[0, :] = jnp.ones((16,), jnp.int32) * ncnk
    pltpu.sync_copy(cnt_v, counts.at[b])


_sc_bucket = pl.kernel(
    _sc_bucket_body,
    out_type=[
        jax.ShapeDtypeStruct((NB * CAP,), jnp.int32),
        jax.ShapeDtypeStruct((NB, 1, 16), jnp.int32),
    ],
    mesh=_MESH,
    scratch_types=[
        pltpu.VMEM((SCHUNK,), jnp.int32),
        pltpu.VMEM((FBUF,), jnp.int32),
        pltpu.VMEM((1, 16), jnp.int32),
    ],
    compiler_params=pltpu.CompilerParams(needs_layout_passes=False),
    name="sc_edge_bucketize",
)


def _sc_agg_body(h_hbm, lists, counts, zacc, out,
                 pk_v, sidx0, sidx1, dloc0, dloc1, rows0, rows1,
                 hist_v, cnt_v, acc, sem0, sem1):
    c = lax.axis_index("c")
    s = lax.axis_index("s")
    b = c * NS + s
    lbase = pl.multiple_of(b * CAP, 8)

    pltpu.sync_copy(counts.at[b], cnt_v)
    ncnk = cnt_v[0, :][0]
    pltpu.sync_copy(zacc, acc)

    def zh(i, carry):
        hist_v[pl.ds(i * 16, 16)] = jnp.zeros((16,), jnp.float32)
        return carry

    lax.fori_loop(0, BR // 16, zh, 0)
    ones16 = jnp.ones((16,), jnp.float32)
    iota16 = lax.iota(jnp.int32, 16)

    def unpack(cc, sidx_v, dloc_v):
        def up(k, carry2):
            p = pk_v[pl.ds(cc * LCH + k * 16, 16)]
            sidx_v[pl.ds(k * 16, 16)] = lax.shift_right_logical(p, 14)
            dl = jnp.minimum(lax.shift_right_logical(p, 5) & 511, BR - 1)
            dloc_v[pl.ds(k * 16, 16)] = dl
            plsc.addupdate_scatter(hist_v, [dl], ones16)
            return carry2

        lax.fori_loop(0, LCH // 16, up, 0)

    def issue(sidx_v, rows_v, sem):
        pltpu.async_copy(h_hbm.at[sidx_v], rows_v, sem)

    def wait(sidx_v, rows_v, sem):
        pltpu.make_async_copy(h_hbm.at[sidx_v], rows_v, sem).wait()

    def accumulate(dloc_v, rows_v):
        def acc_g(g, carry2):
            dv = dloc_v[pl.ds(g * 16, 16)]
            ev = g * 16 + iota16
            for f in range(D):
                ff = jnp.full((16,), f, jnp.int32)
                col = plsc.load_gather(rows_v, [ev, ff])
                plsc.addupdate_scatter(acc, [dv, ff], col)
            return carry2

        lax.fori_loop(0, LCH // 16, acc_g, 0)

    nsb = (ncnk + 15) // 16

    def sb_loop(sb, carry):
        base = pl.multiple_of(lbase + sb * SBE, 8)
        pltpu.sync_copy(lists.at[pl.ds(base, SBE)], pk_v)
        nch = jnp.minimum(ncnk - sb * 16, 16)
        unpack(0, sidx0, dloc0)
        issue(sidx0, rows0, sem0)

        def pair(pp, carry2):
            unpack(2 * pp + 1, sidx1, dloc1)
            issue(sidx1, rows1, sem1)
            wait(sidx0, rows0, sem0)
            accumulate(dloc0, rows0)

            def refill(_):
                unpack(2 * pp + 2, sidx0, dloc0)
                issue(sidx0, rows0, sem0)
                return 0

            lax.cond(2 * pp + 2 < nch, refill, lambda _: 0, 0)
            wait(sidx1, rows1, sem1)
            accumulate(dloc1, rows1)
            return carry2

        lax.fori_loop(0, nch // 2, pair, 0)
        return carry

    lax.fori_loop(0, nsb, sb_loop, 0)

    def divg(g, carry):
        recv = 1.0 / jnp.maximum(hist_v[pl.ds(g * 16, 16)], 1.0)
        r0 = g * 16
        for lane in range(16):
            rv = jnp.ones((16,), jnp.float32) * recv[lane]
            for k in range(8):
                acc[r0 + lane, pl.ds(k * 16, 16)] = (
                    acc[r0 + lane, pl.ds(k * 16, 16)] * rv)
        return carry

    lax.fori_loop(0, BR // 16, divg, 0)
    pltpu.sync_copy(acc, out.at[b])


_sc_agg = pl.kernel(
    _sc_agg_body,
    out_type=[jax.ShapeDtypeStruct((NB, BR, D), jnp.float32)],
    mesh=_MESH,
    scratch_types=[
        pltpu.VMEM((SBE,), jnp.int32),
        pltpu.VMEM((LCH,), jnp.int32),
        pltpu.VMEM((LCH,), jnp.int32),
        pltpu.VMEM((LCH,), jnp.int32),
        pltpu.VMEM((LCH,), jnp.int32),
        pltpu.VMEM((LCH, D), jnp.float32),
        pltpu.VMEM((LCH, D), jnp.float32),
        pltpu.VMEM((BR,), jnp.float32),
        pltpu.VMEM((1, 16), jnp.int32),
        pltpu.VMEM((BR, D), jnp.float32),
        pltpu.SemaphoreType.DMA,
        pltpu.SemaphoreType.DMA,
    ],
    compiler_params=pltpu.CompilerParams(needs_layout_passes=False),
    name="sc_segment_mean",
)


def _row_mask():
    rows = lax.broadcasted_iota(jnp.int32, (NP2, 1), 0)
    return rows < N


def _uninterleave(o):
    return jnp.swapaxes(o, 0, 1).reshape(NP2, D)


def _dense_layer(aggb, h, Wl, bl, Wr, lnw, lnb, a):
    mask = _row_mask()
    agg = _uninterleave(aggb)
    t = (jnp.dot(agg, Wl.T, preferred_element_type=jnp.float32) + bl
         + jnp.dot(h, Wr.T, preferred_element_type=jnp.float32))
    t = jnp.where(mask, t, 0.0)
    denom = float(N * D)
    mu = jnp.sum(t) / denom
    centered = jnp.where(mask, t - mu, 0.0)
    var = jnp.sum(centered * centered) / denom
    out = centered * lax.rsqrt(var + EPS) * lnw + lnb
    out = jnp.where(out > 0, out, a * out)
    return jnp.where(mask, out, 0.0)


def _tc_dense0_body(xp_ref, aggb_ref, Wl_ref, bl_ref, Wr_ref,
                    lnw_ref, lnb_ref, a_ref, Wskip_ref, h1_ref):
    xp = xp_ref[...]
    h0 = _dense_layer(aggb_ref[...], xp, Wl_ref[...], bl_ref[...],
                      Wr_ref[...], lnw_ref[...], lnb_ref[...], a_ref[0, 0])
    h1 = jnp.dot(xp, Wskip_ref[...].T, preferred_element_type=jnp.float32) + h0
    h1_ref[...] = jnp.where(_row_mask(), h1, 0.0)


def _tc_dense1_body(h1_ref, aggb_ref, Wl_ref, bl_ref, Wr_ref,
                    lnw_ref, lnb_ref, a_ref, out_ref):
    out_ref[...] = _dense_layer(aggb_ref[...], h1_ref[...], Wl_ref[...],
                                bl_ref[...], Wr_ref[...], lnw_ref[...],
                                lnb_ref[...], a_ref[0, 0])


_tc_dense0 = pl.pallas_call(
    _tc_dense0_body,
    out_shape=jax.ShapeDtypeStruct((NP2, D), jnp.float32),
)

_tc_dense1 = pl.pallas_call(
    _tc_dense1_body,
    out_shape=jax.ShapeDtypeStruct((NP2, D), jnp.float32),
)


def kernel(x, edge_index, Wl0, bl0, Wr0, lnw0, lnb0, a0, Wskip,
           Wl1, bl1, Wr1, lnw1, lnb1, a1):
    packed = edge_index[1] + (edge_index[0] << 14)
    xp = jnp.pad(x, ((0, NP2 - N), (0, 0)))
    zacc = jnp.zeros((BR, D), jnp.float32)
    bl0r = bl0.reshape(1, D)
    bl1r = bl1.reshape(1, D)
    lnw0r = lnw0.reshape(1, D)
    lnb0r = lnb0.reshape(1, D)
    lnw1r = lnw1.reshape(1, D)
    lnb1r = lnb1.reshape(1, D)
    a0r = a0.reshape(1, 1)
    a1r = a1.reshape(1, 1)

    lists, counts = _sc_bucket(packed)
    (aggb0,) = _sc_agg(xp, lists, counts, zacc)
    h1p = _tc_dense0(xp, aggb0, Wl0, bl0r, Wr0, lnw0r, lnb0r, a0r, Wskip)
    (aggb1,) = _sc_agg(h1p, lists, counts, zacc)
    outp = _tc_dense1(h1p, aggb1, Wl1, bl1r, Wr1, lnw1r, lnb1r, a1r)
    return outp[:N]


# speedup vs baseline: 3.6332x; 3.6332x over previous
import jax
import jax.numpy as jnp
from jax import lax
from jax.experimental import pallas as pl
from jax.experimental.pallas import tpu as pltpu
from jax.experimental.pallas import tpu_sc as plsc

N = 10000
D = 128
E = 320000
EPS = 1e-5

NC = 2
NS = 16
NB = NC * NS
BR = 320
NP2 = BR * NB
SCHUNK = 8000
NSC = E // SCHUNK
FBUF = 8448
CAP = E + 4096
LCH = 128
SBE = 2048
PADBASE = (BR - 1) * NB

_MESH = plsc.VectorSubcoreMesh(core_axis_name="c", subcore_axis_name="s",
                               num_cores=NC, num_subcores=NS)


def _sc_bucket_body(packed, lists, counts, stage_v, fbuf, cnt_v):
    c = lax.axis_index("c")
    s = lax.axis_index("s")
    b = c * NS + s
    lbase = pl.multiple_of(b * CAP, 8)

    def outer(sc, carry):
        off, goff = carry
        pltpu.sync_copy(packed.at[pl.ds(sc * SCHUNK, SCHUNK)], stage_v)

        def scan(i, o):
            p = stage_v[pl.ds(i * 16, 16)]
            msk = (p & 31) == b
            plsc.store_compressed(fbuf.at[pl.ds(o, 16)], p, mask=msk)
            return o + plsc.all_reduce_population_count(msk)[0]

        off = lax.fori_loop(0, SCHUNK // 16, scan, off)
        nf = off // 128
        r0 = nf * 128

        def fl(k, carry2):
            dst0 = pl.multiple_of(lbase + goff + k * 128, 8)
            pltpu.sync_copy(fbuf.at[pl.ds(k * 128, 128)],
                            lists.at[pl.ds(dst0, 128)])
            return carry2

        lax.fori_loop(0, nf, fl, 0)

        def mv(j, carry2):
            v = fbuf[pl.ds(r0 + j * 16, 16)]
            fbuf[pl.ds(j * 16, 16)] = v
            return carry2

        lax.fori_loop(0, 9, mv, 0)
        return (off - r0, goff + r0)

    off, goff = lax.fori_loop(0, NSC, outer, (0, 0))

    padv = jnp.full((16,), PADBASE, jnp.int32) + b
    fullm = jnp.ones((16,), jnp.bool_)

    def padstep(k, o):
        plsc.store_compressed(fbuf.at[pl.ds(o, 16)], padv, mask=fullm)
        return o + 16

    lax.fori_loop(0, 16, padstep, off)
    total = goff // 128 + (off + 127) // 128
    ncnk = total + (total & 1)
    tail_rows = ncnk - goff // 128

    def fflush(k, carry):
        dst0 = pl.multiple_of(lbase + goff + k * 128, 8)
        pltpu.sync_copy(fbuf.at[pl.ds(k * 128, 128)],
                        lists.at[pl.ds(dst0, 128)])
        return carry

    lax.fori_loop(0, tail_rows, fflush, 0)
    cnt_v[0, :] = jnp.ones((16,), jnp.int32) * ncnk
    pltpu.sync_copy(cnt_v, counts.at[b])


_sc_bucket = pl.kernel(
    _sc_bucket_body,
    out_type=[
        jax.ShapeDtypeStruct((NB * CAP,), jnp.int32),
        jax.ShapeDtypeStruct((NB, 1, 16), jnp.int32),
    ],
    mesh=_MESH,
    scratch_types=[
        pltpu.VMEM((SCHUNK,), jnp.int32),
        pltpu.VMEM((FBUF,), jnp.int32),
        pltpu.VMEM((1, 16), jnp.int32),
    ],
    compiler_params=pltpu.CompilerParams(needs_layout_passes=False),
    name="sc_edge_bucketize",
)


def _sc_agg_body(h_hbm, lists, counts, zacc, out,
                 pk_v, sidx0, sidx1, dloc0, dloc1, rows0, rows1,
                 hist_v, cnt_v, acc, sem0, sem1):
    c = lax.axis_index("c")
    s = lax.axis_index("s")
    b = c * NS + s
    lbase = pl.multiple_of(b * CAP, 8)

    pltpu.sync_copy(counts.at[b], cnt_v)
    ncnk = cnt_v[0, :][0]
    pltpu.sync_copy(zacc, acc)

    def zh(i, carry):
        hist_v[pl.ds(i * 16, 16)] = jnp.zeros((16,), jnp.float32)
        return carry

    lax.fori_loop(0, BR // 16, zh, 0)
    ones16 = jnp.ones((16,), jnp.float32)
    iota16 = lax.iota(jnp.int32, 16)

    def unpack(cc, sidx_v, dloc_v):
        def up(k, carry2):
            p = pk_v[pl.ds(cc * LCH + k * 16, 16)]
            sidx_v[pl.ds(k * 16, 16)] = lax.shift_right_logical(p, 14)
            dl = jnp.minimum(lax.shift_right_logical(p, 5) & 511, BR - 1)
            dloc_v[pl.ds(k * 16, 16)] = dl
            plsc.addupdate_scatter(hist_v, [dl], ones16)
            return carry2

        lax.fori_loop(0, LCH // 16, up, 0)

    def issue(sidx_v, rows_v, sem):
        pltpu.async_copy(h_hbm.at[sidx_v], rows_v, sem)

    def wait(sidx_v, rows_v, sem):
        pltpu.make_async_copy(h_hbm.at[sidx_v], rows_v, sem).wait()

    def accumulate(dloc_v, rows_v):
        @plsc.parallel_loop(0, LCH // 16, 1, unroll=2)
        def acc_g(g):
            dv = dloc_v[pl.ds(g * 16, 16)]
            e0 = g * 16
            for lane in range(16):
                dl = dv[lane]
                for k in range(8):
                    plsc.addupdate(acc.at[dl, pl.ds(k * 16, 16)],
                                   rows_v[e0 + lane, pl.ds(k * 16, 16)])

    nsb = (ncnk + 15) // 16

    def sb_loop(sb, carry):
        base = pl.multiple_of(lbase + sb * SBE, 8)
        pltpu.sync_copy(lists.at[pl.ds(base, SBE)], pk_v)
        nch = jnp.minimum(ncnk - sb * 16, 16)
        unpack(0, sidx0, dloc0)
        issue(sidx0, rows0, sem0)

        def pair(pp, carry2):
            unpack(2 * pp + 1, sidx1, dloc1)
            issue(sidx1, rows1, sem1)
            wait(sidx0, rows0, sem0)
            accumulate(dloc0, rows0)

            def refill(_):
                unpack(2 * pp + 2, sidx0, dloc0)
                issue(sidx0, rows0, sem0)
                return 0

            lax.cond(2 * pp + 2 < nch, refill, lambda _: 0, 0)
            wait(sidx1, rows1, sem1)
            accumulate(dloc1, rows1)
            return carry2

        lax.fori_loop(0, nch // 2, pair, 0)
        return carry

    lax.fori_loop(0, nsb, sb_loop, 0)

    def divg(g, carry):
        recv = 1.0 / jnp.maximum(hist_v[pl.ds(g * 16, 16)], 1.0)
        r0 = g * 16
        for lane in range(16):
            rv = jnp.ones((16,), jnp.float32) * recv[lane]
            for k in range(8):
                acc[r0 + lane, pl.ds(k * 16, 16)] = (
                    acc[r0 + lane, pl.ds(k * 16, 16)] * rv)
        return carry

    lax.fori_loop(0, BR // 16, divg, 0)
    pltpu.sync_copy(acc, out.at[b])


_sc_agg = pl.kernel(
    _sc_agg_body,
    out_type=[jax.ShapeDtypeStruct((NB, BR, D), jnp.float32)],
    mesh=_MESH,
    scratch_types=[
        pltpu.VMEM((SBE,), jnp.int32),
        pltpu.VMEM((LCH,), jnp.int32),
        pltpu.VMEM((LCH,), jnp.int32),
        pltpu.VMEM((LCH,), jnp.int32),
        pltpu.VMEM((LCH,), jnp.int32),
        pltpu.VMEM((LCH, D), jnp.float32),
        pltpu.VMEM((LCH, D), jnp.float32),
        pltpu.VMEM((BR,), jnp.float32),
        pltpu.VMEM((1, 16), jnp.int32),
        pltpu.VMEM((BR, D), jnp.float32),
        pltpu.SemaphoreType.DMA,
        pltpu.SemaphoreType.DMA,
    ],
    compiler_params=pltpu.CompilerParams(needs_layout_passes=False),
    name="sc_segment_mean",
)


def _row_mask():
    rows = lax.broadcasted_iota(jnp.int32, (NP2, 1), 0)
    return rows < N


def _uninterleave(o):
    return jnp.swapaxes(o, 0, 1).reshape(NP2, D)


def _dense_layer(aggb, h, Wl, bl, Wr, lnw, lnb, a):
    mask = _row_mask()
    agg = _uninterleave(aggb)
    t = (jnp.dot(agg, Wl.T, preferred_element_type=jnp.float32) + bl
         + jnp.dot(h, Wr.T, preferred_element_type=jnp.float32))
    t = jnp.where(mask, t, 0.0)
    denom = float(N * D)
    mu = jnp.sum(t) / denom
    centered = jnp.where(mask, t - mu, 0.0)
    var = jnp.sum(centered * centered) / denom
    out = centered * lax.rsqrt(var + EPS) * lnw + lnb
    out = jnp.where(out > 0, out, a * out)
    return jnp.where(mask, out, 0.0)


def _tc_dense0_body(xp_ref, aggb_ref, Wl_ref, bl_ref, Wr_ref,
                    lnw_ref, lnb_ref, a_ref, Wskip_ref, h1_ref):
    xp = xp_ref[...]
    h0 = _dense_layer(aggb_ref[...], xp, Wl_ref[...], bl_ref[...],
                      Wr_ref[...], lnw_ref[...], lnb_ref[...], a_ref[0, 0])
    h1 = jnp.dot(xp, Wskip_ref[...].T, preferred_element_type=jnp.float32) + h0
    h1_ref[...] = jnp.where(_row_mask(), h1, 0.0)


def _tc_dense1_body(h1_ref, aggb_ref, Wl_ref, bl_ref, Wr_ref,
                    lnw_ref, lnb_ref, a_ref, out_ref):
    out_ref[...] = _dense_layer(aggb_ref[...], h1_ref[...], Wl_ref[...],
                                bl_ref[...], Wr_ref[...], lnw_ref[...],
                                lnb_ref[...], a_ref[0, 0])


_tc_dense0 = pl.pallas_call(
    _tc_dense0_body,
    out_shape=jax.ShapeDtypeStruct((NP2, D), jnp.float32),
)

_tc_dense1 = pl.pallas_call(
    _tc_dense1_body,
    out_shape=jax.ShapeDtypeStruct((NP2, D), jnp.float32),
)


def kernel(x, edge_index, Wl0, bl0, Wr0, lnw0, lnb0, a0, Wskip,
           Wl1, bl1, Wr1, lnw1, lnb1, a1):
    packed = edge_index[1] + (edge_index[0] << 14)
    xp = jnp.pad(x, ((0, NP2 - N), (0, 0)))
    zacc = jnp.zeros((BR, D), jnp.float32)
    bl0r = bl0.reshape(1, D)
    bl1r = bl1.reshape(1, D)
    lnw0r = lnw0.reshape(1, D)
    lnb0r = lnb0.reshape(1, D)
    lnw1r = lnw1.reshape(1, D)
    lnb1r = lnb1.reshape(1, D)
    a0r = a0.reshape(1, 1)
    a1r = a1.reshape(1, 1)

    lists, counts = _sc_bucket(packed)
    (aggb0,) = _sc_agg(xp, lists, counts, zacc)
    h1p = _tc_dense0(xp, aggb0, Wl0, bl0r, Wr0, lnw0r, lnb0r, a0r, Wskip)
    (aggb1,) = _sc_agg(h1p, lists, counts, zacc)
    outp = _tc_dense1(h1p, aggb1, Wl1, bl1r, Wr1, lnw1r, lnb1r, a1r)
    return outp[:N]


# speedup vs baseline: 4.2247x; 1.1628x over previous
import jax
import jax.numpy as jnp
from jax import lax
from jax.experimental import pallas as pl
from jax.experimental.pallas import tpu as pltpu
from jax.experimental.pallas import tpu_sc as plsc

N = 10000
D = 128
E = 320000
EPS = 1e-5

NC = 2
NS = 16
NB = NC * NS
BR = 320
NP2 = BR * NB
SCHUNK = 8000
NSC = E // SCHUNK
FBUF = 8448
CAP = E + 4096
LCH = 128
SBE = 2048
PADBASE = (BR - 1) * NB

_MESH = plsc.VectorSubcoreMesh(core_axis_name="c", subcore_axis_name="s",
                               num_cores=NC, num_subcores=NS)


def _sc_bucket_body(packed, lists, counts, stage_v, fbuf, cnt_v):
    c = lax.axis_index("c")
    s = lax.axis_index("s")
    b = c * NS + s
    lbase = pl.multiple_of(b * CAP, 8)

    def outer(sc, carry):
        off, goff = carry
        pltpu.sync_copy(packed.at[pl.ds(sc * SCHUNK, SCHUNK)], stage_v)

        @plsc.parallel_loop(0, SCHUNK // 16, 1, unroll=4, carry=off)
        def scan(i, o):
            p = stage_v[pl.ds(i * 16, 16)]
            msk = (p & 31) == b
            plsc.store_compressed(fbuf.at[pl.ds(o, 16)], p, mask=msk)
            return o + plsc.all_reduce_population_count(msk)[0]

        off = scan
        nf = off // 128
        r0 = nf * 128

        def fl(k, carry2):
            dst0 = pl.multiple_of(lbase + goff + k * 128, 8)
            pltpu.sync_copy(fbuf.at[pl.ds(k * 128, 128)],
                            lists.at[pl.ds(dst0, 128)])
            return carry2

        lax.fori_loop(0, nf, fl, 0)

        def mv(j, carry2):
            v = fbuf[pl.ds(r0 + j * 16, 16)]
            fbuf[pl.ds(j * 16, 16)] = v
            return carry2

        lax.fori_loop(0, 9, mv, 0)
        return (off - r0, goff + r0)

    off, goff = lax.fori_loop(0, NSC, outer, (0, 0))

    padv = jnp.full((16,), PADBASE, jnp.int32) + b
    fullm = jnp.ones((16,), jnp.bool_)

    def padstep(k, o):
        plsc.store_compressed(fbuf.at[pl.ds(o, 16)], padv, mask=fullm)
        return o + 16

    lax.fori_loop(0, 16, padstep, off)
    total = goff // 128 + (off + 127) // 128
    ncnk = total + (total & 1)
    tail_rows = ncnk - goff // 128

    def fflush(k, carry):
        dst0 = pl.multiple_of(lbase + goff + k * 128, 8)
        pltpu.sync_copy(fbuf.at[pl.ds(k * 128, 128)],
                        lists.at[pl.ds(dst0, 128)])
        return carry

    lax.fori_loop(0, tail_rows, fflush, 0)
    cnt_v[0, :] = jnp.ones((16,), jnp.int32) * ncnk
    pltpu.sync_copy(cnt_v, counts.at[b])


_sc_bucket = pl.kernel(
    _sc_bucket_body,
    out_type=[
        jax.ShapeDtypeStruct((NB * CAP,), jnp.int32),
        jax.ShapeDtypeStruct((NB, 1, 16), jnp.int32),
    ],
    mesh=_MESH,
    scratch_types=[
        pltpu.VMEM((SCHUNK,), jnp.int32),
        pltpu.VMEM((FBUF,), jnp.int32),
        pltpu.VMEM((1, 16), jnp.int32),
    ],
    compiler_params=pltpu.CompilerParams(needs_layout_passes=False),
    name="sc_edge_bucketize",
)


def _sc_agg_body(h_hbm, lists, counts, zacc, out,
                 pk_v, sidx0, sidx1, dloc0, dloc1, rows0, rows1,
                 hist_v, cnt_v, acc, sem0, sem1):
    c = lax.axis_index("c")
    s = lax.axis_index("s")
    b = c * NS + s
    lbase = pl.multiple_of(b * CAP, 8)

    pltpu.sync_copy(counts.at[b], cnt_v)
    ncnk = cnt_v[0, :][0]
    pltpu.sync_copy(zacc, acc)

    def zh(i, carry):
        hist_v[pl.ds(i * 16, 16)] = jnp.zeros((16,), jnp.float32)
        return carry

    lax.fori_loop(0, BR // 16, zh, 0)
    ones16 = jnp.ones((16,), jnp.float32)
    iota16 = lax.iota(jnp.int32, 16)

    def unpack(cc, sidx_v, dloc_v):
        def up(k, carry2):
            p = pk_v[pl.ds(cc * LCH + k * 16, 16)]
            sidx_v[pl.ds(k * 16, 16)] = lax.shift_right_logical(p, 14)
            dl = jnp.minimum(lax.shift_right_logical(p, 5) & 511, BR - 1)
            dloc_v[pl.ds(k * 16, 16)] = dl
            plsc.addupdate_scatter(hist_v, [dl], ones16)
            return carry2

        lax.fori_loop(0, LCH // 16, up, 0)

    def issue(sidx_v, rows_v, sem):
        pltpu.async_copy(h_hbm.at[sidx_v], rows_v, sem)

    def wait(sidx_v, rows_v, sem):
        pltpu.make_async_copy(h_hbm.at[sidx_v], rows_v, sem).wait()

    def accumulate(dloc_v, rows_v):
        @plsc.parallel_loop(0, LCH // 16, 1, unroll=4)
        def acc_g(g):
            dv = dloc_v[pl.ds(g * 16, 16)]
            e0 = g * 16
            for lane in range(16):
                dl = dv[lane]
                for k in range(8):
                    plsc.addupdate(acc.at[dl, pl.ds(k * 16, 16)],
                                   rows_v[e0 + lane, pl.ds(k * 16, 16)])

    nsb = (ncnk + 15) // 16

    def sb_loop(sb, carry):
        base = pl.multiple_of(lbase + sb * SBE, 8)
        pltpu.sync_copy(lists.at[pl.ds(base, SBE)], pk_v)
        nch = jnp.minimum(ncnk - sb * 16, 16)
        unpack(0, sidx0, dloc0)
        issue(sidx0, rows0, sem0)

        def pair(pp, carry2):
            unpack(2 * pp + 1, sidx1, dloc1)
            issue(sidx1, rows1, sem1)
            wait(sidx0, rows0, sem0)
            accumulate(dloc0, rows0)

            def refill(_):
                unpack(2 * pp + 2, sidx0, dloc0)
                issue(sidx0, rows0, sem0)
                return 0

            lax.cond(2 * pp + 2 < nch, refill, lambda _: 0, 0)
            wait(sidx1, rows1, sem1)
            accumulate(dloc1, rows1)
            return carry2

        lax.fori_loop(0, nch // 2, pair, 0)
        return carry

    lax.fori_loop(0, nsb, sb_loop, 0)

    def divg(g, carry):
        recv = 1.0 / jnp.maximum(hist_v[pl.ds(g * 16, 16)], 1.0)
        r0 = g * 16
        for lane in range(16):
            rv = jnp.ones((16,), jnp.float32) * recv[lane]
            for k in range(8):
                acc[r0 + lane, pl.ds(k * 16, 16)] = (
                    acc[r0 + lane, pl.ds(k * 16, 16)] * rv)
        return carry

    lax.fori_loop(0, BR // 16, divg, 0)
    pltpu.sync_copy(acc, out.at[b])


_sc_agg = pl.kernel(
    _sc_agg_body,
    out_type=[jax.ShapeDtypeStruct((NB, BR, D), jnp.float32)],
    mesh=_MESH,
    scratch_types=[
        pltpu.VMEM((SBE,), jnp.int32),
        pltpu.VMEM((LCH,), jnp.int32),
        pltpu.VMEM((LCH,), jnp.int32),
        pltpu.VMEM((LCH,), jnp.int32),
        pltpu.VMEM((LCH,), jnp.int32),
        pltpu.VMEM((LCH, D), jnp.float32),
        pltpu.VMEM((LCH, D), jnp.float32),
        pltpu.VMEM((BR,), jnp.float32),
        pltpu.VMEM((1, 16), jnp.int32),
        pltpu.VMEM((BR, D), jnp.float32),
        pltpu.SemaphoreType.DMA,
        pltpu.SemaphoreType.DMA,
    ],
    compiler_params=pltpu.CompilerParams(needs_layout_passes=False),
    name="sc_segment_mean",
)


def _row_mask():
    rows = lax.broadcasted_iota(jnp.int32, (NP2, 1), 0)
    return rows < N


def _uninterleave(o):
    return jnp.swapaxes(o, 0, 1).reshape(NP2, D)


def _dense_layer(aggb, h, Wl, bl, Wr, lnw, lnb, a):
    mask = _row_mask()
    agg = _uninterleave(aggb)
    t = (jnp.dot(agg, Wl.T, preferred_element_type=jnp.float32) + bl
         + jnp.dot(h, Wr.T, preferred_element_type=jnp.float32))
    t = jnp.where(mask, t, 0.0)
    denom = float(N * D)
    mu = jnp.sum(t) / denom
    centered = jnp.where(mask, t - mu, 0.0)
    var = jnp.sum(centered * centered) / denom
    out = centered * lax.rsqrt(var + EPS) * lnw + lnb
    out = jnp.where(out > 0, out, a * out)
    return jnp.where(mask, out, 0.0)


def _tc_dense0_body(xp_ref, aggb_ref, Wl_ref, bl_ref, Wr_ref,
                    lnw_ref, lnb_ref, a_ref, Wskip_ref, h1_ref):
    xp = xp_ref[...]
    h0 = _dense_layer(aggb_ref[...], xp, Wl_ref[...], bl_ref[...],
                      Wr_ref[...], lnw_ref[...], lnb_ref[...], a_ref[0, 0])
    h1 = jnp.dot(xp, Wskip_ref[...].T, preferred_element_type=jnp.float32) + h0
    h1_ref[...] = jnp.where(_row_mask(), h1, 0.0)


def _tc_dense1_body(h1_ref, aggb_ref, Wl_ref, bl_ref, Wr_ref,
                    lnw_ref, lnb_ref, a_ref, out_ref):
    out_ref[...] = _dense_layer(aggb_ref[...], h1_ref[...], Wl_ref[...],
                                bl_ref[...], Wr_ref[...], lnw_ref[...],
                                lnb_ref[...], a_ref[0, 0])


_tc_dense0 = pl.pallas_call(
    _tc_dense0_body,
    out_shape=jax.ShapeDtypeStruct((NP2, D), jnp.float32),
)

_tc_dense1 = pl.pallas_call(
    _tc_dense1_body,
    out_shape=jax.ShapeDtypeStruct((NP2, D), jnp.float32),
)


def kernel(x, edge_index, Wl0, bl0, Wr0, lnw0, lnb0, a0, Wskip,
           Wl1, bl1, Wr1, lnw1, lnb1, a1):
    packed = edge_index[1] + (edge_index[0] << 14)
    xp = jnp.pad(x, ((0, NP2 - N), (0, 0)))
    zacc = jnp.zeros((BR, D), jnp.float32)
    bl0r = bl0.reshape(1, D)
    bl1r = bl1.reshape(1, D)
    lnw0r = lnw0.reshape(1, D)
    lnb0r = lnb0.reshape(1, D)
    lnw1r = lnw1.reshape(1, D)
    lnb1r = lnb1.reshape(1, D)
    a0r = a0.reshape(1, 1)
    a1r = a1.reshape(1, 1)

    lists, counts = _sc_bucket(packed)
    (aggb0,) = _sc_agg(xp, lists, counts, zacc)
    h1p = _tc_dense0(xp, aggb0, Wl0, bl0r, Wr0, lnw0r, lnb0r, a0r, Wskip)
    (aggb1,) = _sc_agg(h1p, lists, counts, zacc)
    outp = _tc_dense1(h1p, aggb1, Wl1, bl1r, Wr1, lnw1r, lnb1r, a1r)
    return outp[:N]


# speedup vs baseline: 5.9080x; 1.3984x over previous
import jax
import jax.numpy as jnp
from jax import lax
from jax.experimental import pallas as pl
from jax.experimental.pallas import tpu as pltpu
from jax.experimental.pallas import tpu_sc as plsc

N = 10000
D = 128
E = 320000
EPS = 1e-5

NC = 2
NS = 16
NB = NC * NS
BR = 320
NP2 = BR * NB
SCHUNK = 8000
NSC = E // SCHUNK
FBUF = 8448
CAP = E + 4096
LCH = 128
SBE = 2048
PADBASE = (BR - 1) * NB

_MESH = plsc.VectorSubcoreMesh(core_axis_name="c", subcore_axis_name="s",
                               num_cores=NC, num_subcores=NS)


def _sc_bucket_body(packed, lists, counts, stage_v, fbuf, cnt_v):
    c = lax.axis_index("c")
    s = lax.axis_index("s")
    b = c * NS + s
    lbase = pl.multiple_of(b * CAP, 8)

    def outer(sc, carry):
        off, goff = carry
        pltpu.sync_copy(packed.at[pl.ds(sc * SCHUNK, SCHUNK)], stage_v)

        @plsc.parallel_loop(0, SCHUNK // 16, 1, unroll=4, carry=off)
        def scan(i, o):
            p = stage_v[pl.ds(i * 16, 16)]
            msk = (p & 31) == b
            plsc.store_compressed(fbuf.at[pl.ds(o, 16)], p, mask=msk)
            return o + plsc.all_reduce_population_count(msk)[0]

        off = scan
        nf = off // 128
        r0 = nf * 128

        def fl(k, carry2):
            dst0 = pl.multiple_of(lbase + goff + k * 128, 8)
            pltpu.sync_copy(fbuf.at[pl.ds(k * 128, 128)],
                            lists.at[pl.ds(dst0, 128)])
            return carry2

        lax.fori_loop(0, nf, fl, 0)

        def mv(j, carry2):
            v = fbuf[pl.ds(r0 + j * 16, 16)]
            fbuf[pl.ds(j * 16, 16)] = v
            return carry2

        lax.fori_loop(0, 9, mv, 0)
        return (off - r0, goff + r0)

    off, goff = lax.fori_loop(0, NSC, outer, (0, 0))

    padv = jnp.full((16,), PADBASE, jnp.int32) + b
    fullm = jnp.ones((16,), jnp.bool_)

    def padstep(k, o):
        plsc.store_compressed(fbuf.at[pl.ds(o, 16)], padv, mask=fullm)
        return o + 16

    lax.fori_loop(0, 16, padstep, off)
    total = goff // 128 + (off + 127) // 128
    ncnk = total + (total & 1)
    tail_rows = ncnk - goff // 128

    def fflush(k, carry):
        dst0 = pl.multiple_of(lbase + goff + k * 128, 8)
        pltpu.sync_copy(fbuf.at[pl.ds(k * 128, 128)],
                        lists.at[pl.ds(dst0, 128)])
        return carry

    lax.fori_loop(0, tail_rows, fflush, 0)
    cnt_v[0, :] = jnp.ones((16,), jnp.int32) * ncnk
    pltpu.sync_copy(cnt_v, counts.at[b])


_sc_bucket = pl.kernel(
    _sc_bucket_body,
    out_type=[
        jax.ShapeDtypeStruct((NB * CAP,), jnp.int32),
        jax.ShapeDtypeStruct((NB, 1, 16), jnp.int32),
    ],
    mesh=_MESH,
    scratch_types=[
        pltpu.VMEM((SCHUNK,), jnp.int32),
        pltpu.VMEM((FBUF,), jnp.int32),
        pltpu.VMEM((1, 16), jnp.int32),
    ],
    compiler_params=pltpu.CompilerParams(needs_layout_passes=False),
    name="sc_edge_bucketize",
)


def _sc_agg_body(h_hbm, lists, counts, zacc, out,
                 pk_v, sidx0, sidx1, dloc0, dloc1, rows0, rows1,
                 hist_v, cnt_v, accsh, sem0, sem1):
    c = lax.axis_index("c")
    s = lax.axis_index("s")
    b = c * NS + s
    lbase = pl.multiple_of(b * CAP, 8)
    arow0 = pl.multiple_of(s * BR, 8)

    pltpu.sync_copy(counts.at[b], cnt_v)
    ncnk = cnt_v[0, :][0]
    pltpu.sync_copy(zacc, accsh.at[pl.ds(arow0, BR)])

    def zh(i, carry):
        hist_v[pl.ds(i * 16, 16)] = jnp.zeros((16,), jnp.float32)
        return carry

    lax.fori_loop(0, BR // 16, zh, 0)
    ones16 = jnp.ones((16,), jnp.float32)

    def unpack(cc, sidx_v, dloc_v):
        def up(k, carry2):
            p = pk_v[pl.ds(cc * LCH + k * 16, 16)]
            sidx_v[pl.ds(k * 16, 16)] = lax.shift_right_logical(p, 14)
            dl = jnp.minimum(lax.shift_right_logical(p, 5) & 511, BR - 1)
            plsc.addupdate_scatter(hist_v, [dl], ones16)
            dloc_v[pl.ds(k * 16, 16)] = dl + arow0
            return carry2

        lax.fori_loop(0, LCH // 16, up, 0)

    def issue(sidx_v, rows_v, sem):
        pltpu.async_copy(h_hbm.at[sidx_v], rows_v, sem)

    def wait(sidx_v, rows_v, sem):
        pltpu.make_async_copy(h_hbm.at[sidx_v], rows_v, sem).wait()

    def accumulate(dloc_v, rows_v):
        pltpu.sync_copy(rows_v, accsh.at[dloc_v], add=True)

    nsb = (ncnk + 15) // 16

    def sb_loop(sb, carry):
        base = pl.multiple_of(lbase + sb * SBE, 8)
        pltpu.sync_copy(lists.at[pl.ds(base, SBE)], pk_v)
        nch = jnp.minimum(ncnk - sb * 16, 16)
        unpack(0, sidx0, dloc0)
        issue(sidx0, rows0, sem0)

        def pair(pp, carry2):
            unpack(2 * pp + 1, sidx1, dloc1)
            issue(sidx1, rows1, sem1)
            wait(sidx0, rows0, sem0)
            accumulate(dloc0, rows0)

            def refill(_):
                unpack(2 * pp + 2, sidx0, dloc0)
                issue(sidx0, rows0, sem0)
                return 0

            lax.cond(2 * pp + 2 < nch, refill, lambda _: 0, 0)
            wait(sidx1, rows1, sem1)
            accumulate(dloc1, rows1)
            return carry2

        lax.fori_loop(0, nch // 2, pair, 0)
        return carry

    lax.fori_loop(0, nsb, sb_loop, 0)

    for t in range(BR // 64):
        blk = rows0.at[pl.ds(0, 64)]
        pltpu.sync_copy(accsh.at[pl.ds(arow0 + t * 64, 64)], blk)

        @plsc.parallel_loop(0, 4, 1, unroll=2)
        def divg(g):
            recv = 1.0 / jnp.maximum(hist_v[pl.ds(t * 64 + g * 16, 16)], 1.0)
            r0 = g * 16
            for lane in range(16):
                rv = jnp.ones((16,), jnp.float32) * recv[lane]
                for k in range(8):
                    rows0[r0 + lane, pl.ds(k * 16, 16)] = (
                        rows0[r0 + lane, pl.ds(k * 16, 16)] * rv)

        pltpu.sync_copy(blk, out.at[b, pl.ds(t * 64, 64)])


_sc_agg = pl.kernel(
    _sc_agg_body,
    out_type=[jax.ShapeDtypeStruct((NB, BR, D), jnp.float32)],
    mesh=_MESH,
    scratch_types=[
        pltpu.VMEM((SBE,), jnp.int32),
        pltpu.VMEM((LCH,), jnp.int32),
        pltpu.VMEM((LCH,), jnp.int32),
        pltpu.VMEM((LCH,), jnp.int32),
        pltpu.VMEM((LCH,), jnp.int32),
        pltpu.VMEM((LCH, D), jnp.float32),
        pltpu.VMEM((LCH, D), jnp.float32),
        pltpu.VMEM((BR,), jnp.float32),
        pltpu.VMEM((1, 16), jnp.int32),
        pltpu.VMEM_SHARED((NS * BR, D), jnp.float32),
        pltpu.SemaphoreType.DMA,
        pltpu.SemaphoreType.DMA,
    ],
    compiler_params=pltpu.CompilerParams(needs_layout_passes=False),
    name="sc_segment_mean",
)


def _row_mask():
    rows = lax.broadcasted_iota(jnp.int32, (NP2, 1), 0)
    return rows < N


def _uninterleave(o):
    return jnp.swapaxes(o, 0, 1).reshape(NP2, D)


def _dense_layer(aggb, h, Wl, bl, Wr, lnw, lnb, a):
    mask = _row_mask()
    agg = _uninterleave(aggb)
    t = (jnp.dot(agg, Wl.T, preferred_element_type=jnp.float32) + bl
         + jnp.dot(h, Wr.T, preferred_element_type=jnp.float32))
    t = jnp.where(mask, t, 0.0)
    denom = float(N * D)
    mu = jnp.sum(t) / denom
    centered = jnp.where(mask, t - mu, 0.0)
    var = jnp.sum(centered * centered) / denom
    out = centered * lax.rsqrt(var + EPS) * lnw + lnb
    out = jnp.where(out > 0, out, a * out)
    return jnp.where(mask, out, 0.0)


def _tc_dense0_body(xp_ref, aggb_ref, Wl_ref, bl_ref, Wr_ref,
                    lnw_ref, lnb_ref, a_ref, Wskip_ref, h1_ref):
    xp = xp_ref[...]
    h0 = _dense_layer(aggb_ref[...], xp, Wl_ref[...], bl_ref[...],
                      Wr_ref[...], lnw_ref[...], lnb_ref[...], a_ref[0, 0])
    h1 = jnp.dot(xp, Wskip_ref[...].T, preferred_element_type=jnp.float32) + h0
    h1_ref[...] = jnp.where(_row_mask(), h1, 0.0)


def _tc_dense1_body(h1_ref, aggb_ref, Wl_ref, bl_ref, Wr_ref,
                    lnw_ref, lnb_ref, a_ref, out_ref):
    out_ref[...] = _dense_layer(aggb_ref[...], h1_ref[...], Wl_ref[...],
                                bl_ref[...], Wr_ref[...], lnw_ref[...],
                                lnb_ref[...], a_ref[0, 0])


_tc_dense0 = pl.pallas_call(
    _tc_dense0_body,
    out_shape=jax.ShapeDtypeStruct((NP2, D), jnp.float32),
)

_tc_dense1 = pl.pallas_call(
    _tc_dense1_body,
    out_shape=jax.ShapeDtypeStruct((NP2, D), jnp.float32),
)


def kernel(x, edge_index, Wl0, bl0, Wr0, lnw0, lnb0, a0, Wskip,
           Wl1, bl1, Wr1, lnw1, lnb1, a1):
    packed = edge_index[1] + (edge_index[0] << 14)
    xp = jnp.pad(x, ((0, NP2 - N), (0, 0)))
    zacc = jnp.zeros((BR, D), jnp.float32)
    bl0r = bl0.reshape(1, D)
    bl1r = bl1.reshape(1, D)
    lnw0r = lnw0.reshape(1, D)
    lnb0r = lnb0.reshape(1, D)
    lnw1r = lnw1.reshape(1, D)
    lnb1r = lnb1.reshape(1, D)
    a0r = a0.reshape(1, 1)
    a1r = a1.reshape(1, 1)

    lists, counts = _sc_bucket(packed)
    (aggb0,) = _sc_agg(xp, lists, counts, zacc)
    h1p = _tc_dense0(xp, aggb0, Wl0, bl0r, Wr0, lnw0r, lnb0r, a0r, Wskip)
    (aggb1,) = _sc_agg(h1p, lists, counts, zacc)
    outp = _tc_dense1(h1p, aggb1, Wl1, bl1r, Wr1, lnw1r, lnb1r, a1r)
    return outp[:N]
